# baseline (speedup 1.0000x reference)
import functools

import jax
import jax.numpy as jnp
from jax import lax
from jax.experimental import pallas as pl
from jax.experimental.pallas import tpu as pltpu
from jax.experimental.pallas import tpu_sc as plsc

_BATCH = 1024
_SEQ = 200
_DIM = 64
_NC = 2
_NS = 16
_NW = _NC * _NS
_ROWS_PER_W = _BATCH * _SEQ // _NW
_CHUNK = 128
_NCHUNK = _ROWS_PER_W // _CHUNK


def _gather_kernel(x_hbm, tok_hbm, out_hbm, idx_v, buf0, buf1, sem0, sem1):
    c = lax.axis_index("c")
    s = lax.axis_index("s")
    wid = s * _NC + c
    pltpu.sync_copy(x_hbm.at[wid], idx_v)

    bufs = (buf0, buf1)
    sems = (sem0, sem1)

    pltpu.async_copy(tok_hbm.at[idx_v.at[0]], bufs[0], sems[0])

    def process(g, buf, sem):
        pltpu.make_async_copy(tok_hbm.at[idx_v.at[g]], buf, sem).wait()
        pltpu.sync_copy(buf, out_hbm.at[pl.ds(wid * _ROWS_PER_W + g * _CHUNK,
                                              _CHUNK)])

    def chunk_pair(i, carry):
        g = 2 * i
        pltpu.async_copy(tok_hbm.at[idx_v.at[g + 1]], bufs[1], sems[1])
        process(g, bufs[0], sems[0])

        @pl.when(g + 2 < _NCHUNK)
        def _():
            pltpu.async_copy(tok_hbm.at[idx_v.at[g + 2]], bufs[0], sems[0])

        @pl.when(g + 1 < _NCHUNK)
        def _():
            process(g + 1, bufs[1], sems[1])
        return carry

    lax.fori_loop(0, (_NCHUNK + 1) // 2, chunk_pair, 0)


def kernel(x, token_table, pos_table):
    xi = x.astype(jnp.int32)
    x3 = (xi // 2).reshape(_NW, _NCHUNK, _CHUNK)
    mesh = plsc.VectorSubcoreMesh(core_axis_name="c", subcore_axis_name="s",
                                  num_cores=_NC, num_subcores=_NS)
    run = functools.partial(
        pl.kernel,
        out_type=jax.ShapeDtypeStruct((_BATCH * _SEQ, 2 * _DIM), jnp.float32),
        mesh=mesh,
        compiler_params=pltpu.CompilerParams(use_tc_tiling_on_sc=True),
        scratch_types=[
            pltpu.VMEM((_NCHUNK, _CHUNK), jnp.int32),
            pltpu.VMEM((_CHUNK, 2 * _DIM), jnp.float32),
            pltpu.VMEM((_CHUNK, 2 * _DIM), jnp.float32),
            pltpu.SemaphoreType.DMA,
            pltpu.SemaphoreType.DMA,
        ],
    )(_gather_kernel)
    t2 = token_table.reshape(500000, 2 * _DIM)
    wide = run(x3, t2).reshape(_BATCH, _SEQ, 2, _DIM)
    m = (xi & 1).astype(jnp.float32)[..., None]
    tok_emb = wide[:, :, 0, :] * (1.0 - m) + wide[:, :, 1, :] * m
    return tok_emb + pos_table[None, :_SEQ, :]


# speedup vs baseline: 1.4542x; 1.4542x over previous
import functools

import jax
import jax.numpy as jnp
from jax import lax
from jax.experimental import pallas as pl
from jax.experimental.pallas import tpu as pltpu
from jax.experimental.pallas import tpu_sc as plsc

_BATCH = 1024
_SEQ = 200
_DIM = 64
_NC = 2
_NS = 16
_NW = _NC * _NS
_ROWS_PER_W = _BATCH * _SEQ // _NW
_CHUNK = 640
_NCHUNK = _ROWS_PER_W // _CHUNK


def _gather_kernel(x_hbm, tok_hbm, out_hbm, idx_v, buf0, buf1, sem0, sem1):
    c = lax.axis_index("c")
    s = lax.axis_index("s")
    wid = s * _NC + c
    pltpu.sync_copy(x_hbm.at[wid], idx_v)

    bufs = (buf0, buf1)
    sems = (sem0, sem1)

    pltpu.async_copy(tok_hbm.at[idx_v.at[0]], bufs[0], sems[0])

    def process(g, buf, sem):
        pltpu.make_async_copy(tok_hbm.at[idx_v.at[g]], buf, sem).wait()
        pltpu.sync_copy(buf, out_hbm.at[pl.ds(wid * _ROWS_PER_W + g * _CHUNK,
                                              _CHUNK)])

    def chunk_pair(i, carry):
        g = 2 * i
        pltpu.async_copy(tok_hbm.at[idx_v.at[g + 1]], bufs[1], sems[1])
        process(g, bufs[0], sems[0])

        @pl.when(g + 2 < _NCHUNK)
        def _():
            pltpu.async_copy(tok_hbm.at[idx_v.at[g + 2]], bufs[0], sems[0])
        process(g + 1, bufs[1], sems[1])
        return carry

    lax.fori_loop(0, _NCHUNK // 2, chunk_pair, 0)


def kernel(x, token_table, pos_table):
    x3 = x.astype(jnp.int32).reshape(_NW, _NCHUNK, _CHUNK)
    mesh = plsc.VectorSubcoreMesh(core_axis_name="c", subcore_axis_name="s",
                                  num_cores=_NC, num_subcores=_NS)
    run = functools.partial(
        pl.kernel,
        out_type=jax.ShapeDtypeStruct((_BATCH * _SEQ, _DIM), jnp.float32),
        mesh=mesh,
        compiler_params=pltpu.CompilerParams(use_tc_tiling_on_sc=False),
        scratch_types=[
            pltpu.VMEM((_NCHUNK, _CHUNK), jnp.int32),
            pltpu.VMEM((_CHUNK, _DIM), jnp.float32),
            pltpu.VMEM((_CHUNK, _DIM), jnp.float32),
            pltpu.SemaphoreType.DMA,
            pltpu.SemaphoreType.DMA,
        ],
    )(_gather_kernel)
    tok_emb = run(x3, token_table)
    return tok_emb.reshape(_BATCH, _SEQ, _DIM) + pos_table[None, :_SEQ, :]
